# baseline (speedup 1.0000x reference)
import functools

import jax
import jax.numpy as jnp
from jax import lax
from jax.experimental import pallas as pl
from jax.experimental.pallas import tpu as pltpu
from jax.experimental.pallas import tpu_sc as plsc

_N = 10000
_E = 160000
_G = 64
_NCLS = 12
_NC = 2
_NS = 16
_NW = _NC * _NS
_CHUNK = 128
_CPW = 40
_EPAD = _CHUNK * _NW * _CPW
_RPT0 = 632
_RPTL = _N - (_NS - 1) * _RPT0

_R = 2000
_NB = _N // _R


@functools.lru_cache(None)
def _edge_aggregate(C: int):
  mesh = plsc.VectorSubcoreMesh(core_axis_name="c", subcore_axis_name="s")

  @functools.partial(
      pl.kernel,
      mesh=mesh,
      compiler_params=pltpu.CompilerParams(use_tc_tiling_on_sc=False),
      out_type=jax.ShapeDtypeStruct((_NC, _N, C), jnp.float32),
      scratch_types=[
          pltpu.VMEM((_CPW, _CHUNK), jnp.int32),
          pltpu.VMEM((_CPW, _CHUNK), jnp.int32),
          pltpu.VMEM((_CPW * _CHUNK,), jnp.float32),
          pltpu.VMEM((_CHUNK, C), jnp.float32),
          pltpu.VMEM((_CHUNK, C), jnp.float32),
          pltpu.VMEM((_CHUNK, C), jnp.float32),
          pltpu.VMEM((_CHUNK, C), jnp.float32),
          pltpu.VMEM_SHARED((_N, C), jnp.float32),
          pltpu.SemaphoreType.DMA,
          pltpu.SemaphoreType.DMA,
          pltpu.SemaphoreType.DMA,
          pltpu.SemaphoreType.DMA,
          pltpu.SemaphoreType.DMA,
          pltpu.SemaphoreType.DMA,
          pltpu.SemaphoreType.DMA,
          pltpu.SemaphoreType.DMA,
          pltpu.SemaphoreType.DMA,
      ],
  )
  def agg_kernel(g_hbm, src_hbm, dst_hbm, w_hbm, zero_hbm, out_hbm,
                 src_v, dst_v, w_v, rows_0, rows_1, rows_2, rows_3, acc_sp,
                 sem_g0, sem_g1, sem_g2, sem_g3,
                 sem_s0, sem_s1, sem_s2, sem_s3, sem_ix):
    rows = [rows_0, rows_1, rows_2, rows_3]
    sem_g = [sem_g0, sem_g1, sem_g2, sem_g3]
    sem_s = [sem_s0, sem_s1, sem_s2, sem_s3]
    core = lax.axis_index("c")
    sub = lax.axis_index("s")
    wid = sub * _NC + core
    cbase = wid * _CPW
    pltpu.async_copy(src_hbm.at[pl.ds(cbase, _CPW)], src_v, sem_ix)
    pltpu.async_copy(dst_hbm.at[pl.ds(cbase, _CPW)], dst_v, sem_ix)
    pltpu.async_copy(w_hbm.at[pl.ds(cbase * _CHUNK, _CPW * _CHUNK)], w_v,
                     sem_ix)
    start = pl.multiple_of(sub * _RPT0, 8)

    @pl.when(sub < _NS - 1)
    def _clr_main():
      pltpu.sync_copy(zero_hbm.at[pl.ds(start, _RPT0)],
                      acc_sp.at[pl.ds(start, _RPT0)])

    @pl.when(sub == _NS - 1)
    def _clr_tail():
      pltpu.sync_copy(zero_hbm.at[pl.ds(start, _RPTL)],
                      acc_sp.at[pl.ds(start, _RPTL)])

    pltpu.make_async_copy(src_hbm.at[pl.ds(cbase, _CPW)], src_v, sem_ix).wait()
    pltpu.make_async_copy(dst_hbm.at[pl.ds(cbase, _CPW)], dst_v, sem_ix).wait()
    pltpu.make_async_copy(w_hbm.at[pl.ds(cbase * _CHUNK, _CPW * _CHUNK)],
                          w_v, sem_ix).wait()
    plsc.subcore_barrier()

    gd = lax.GatherDimensionNumbers(offset_dims=(), collapsed_slice_dims=(0,),
                                    start_index_map=(0,))

    def step(c, b):
      pltpu.make_async_copy(g_hbm.at[src_v.at[c]], rows[b], sem_g[b]).wait()
      wbase = pl.multiple_of(c * _CHUNK, _CHUNK)

      def scale_grp(j, carry):
        w16 = w_v[pl.ds(wbase + j * 16, 16)]
        for l in range(16):
          e = j * 16 + l
          wspl = lax.gather(w16, jnp.full((16, 1), l, jnp.int32), gd,
                            slice_sizes=(1,),
                            mode=lax.GatherScatterMode.PROMISE_IN_BOUNDS)
          for cb in range(C // 16):
            sl = pl.ds(cb * 16, 16)
            rows[b][e, sl] = rows[b][e, sl] * wspl
        return carry

      lax.fori_loop(0, _CHUNK // 16, scale_grp, 0, unroll=True)
      pltpu.async_copy(rows[b], acc_sp.at[dst_v.at[c]], sem_s[b], add=True)
      bn = (b + 3) % 4

      @pl.when(c > 0)
      def _():
        pltpu.make_async_copy(rows[bn], acc_sp.at[dst_v.at[c - 1]],
                              sem_s[bn]).wait()

      @pl.when(c + 3 < _CPW)
      def _():
        pltpu.async_copy(g_hbm.at[src_v.at[c + 3]], rows[bn], sem_g[bn])

    for b in range(3):
      pltpu.async_copy(g_hbm.at[src_v.at[b]], rows[b], sem_g[b])

    def run_quad(q, carry):
      for b in range(4):
        step(4 * q + b, b)
      return carry

    lax.fori_loop(0, _CPW // 4, run_quad, 0)
    pltpu.make_async_copy(rows[3], acc_sp.at[dst_v.at[_CPW - 1]],
                          sem_s[3]).wait()
    plsc.subcore_barrier()

    @pl.when(sub < _NS - 1)
    def _wb_main():
      pltpu.sync_copy(acc_sp.at[pl.ds(start, _RPT0)],
                      out_hbm.at[core, pl.ds(start, _RPT0)])

    @pl.when(sub == _NS - 1)
    def _wb_tail():
      pltpu.sync_copy(acc_sp.at[pl.ds(start, _RPTL)],
                      out_hbm.at[core, pl.ds(start, _RPTL)])

  return agg_kernel


def _proj_first(x, w_rel, b_rel, w_root):
  cin = x.shape[1]
  cout = w_rel.shape[0]
  wcat = jnp.concatenate([w_rel, w_root], axis=0)

  def body(x_ref, w_ref, b_ref, g_ref, r_ref):
    h = x_ref[...]
    gr = jnp.dot(h, w_ref[...].T, preferred_element_type=jnp.float32)
    g_ref[...] = gr[:, :cout]
    r_ref[...] = gr[:, cout:] + b_ref[...]

  return pl.pallas_call(
      body,
      grid=(_NB,),
      in_specs=[
          pl.BlockSpec((_R, cin), lambda i: (i, 0)),
          pl.BlockSpec((2 * cout, cin), lambda i: (0, 0)),
          pl.BlockSpec((1, cout), lambda i: (0, 0)),
      ],
      out_specs=[
          pl.BlockSpec((_R, cout), lambda i: (i, 0)),
          pl.BlockSpec((_R, cout), lambda i: (i, 0)),
      ],
      out_shape=[
          jax.ShapeDtypeStruct((_N, cout), jnp.float32),
          jax.ShapeDtypeStruct((_N, cout), jnp.float32),
      ],
  )(x, wcat, b_rel.reshape(1, -1))


def _proj_mid(aggp, r_prev, w_rel, b_rel, w_root):
  cin = r_prev.shape[1]
  cout = w_rel.shape[0]
  wcat = jnp.concatenate([w_rel, w_root], axis=0)

  def body(a_ref, rp_ref, w_ref, b_ref, g_ref, r_ref):
    h = jnp.maximum(a_ref[0] + a_ref[1] + rp_ref[...], 0.0)
    gr = jnp.dot(h, w_ref[...].T, preferred_element_type=jnp.float32)
    g_ref[...] = gr[:, :cout]
    r_ref[...] = gr[:, cout:] + b_ref[...]

  return pl.pallas_call(
      body,
      grid=(_NB,),
      in_specs=[
          pl.BlockSpec((_NC, _R, cin), lambda i: (0, i, 0)),
          pl.BlockSpec((_R, cin), lambda i: (i, 0)),
          pl.BlockSpec((2 * cout, cin), lambda i: (0, 0)),
          pl.BlockSpec((1, cout), lambda i: (0, 0)),
      ],
      out_specs=[
          pl.BlockSpec((_R, cout), lambda i: (i, 0)),
          pl.BlockSpec((_R, cout), lambda i: (i, 0)),
      ],
      out_shape=[
          jax.ShapeDtypeStruct((_N, cout), jnp.float32),
          jax.ShapeDtypeStruct((_N, cout), jnp.float32),
      ],
  )(aggp, r_prev, wcat, b_rel.reshape(1, -1))


def _pool_and_heads(aggp, r_prev, batch3, w1s, b1s, w2s, b2s, w3s, b3s,
                    wos, bos):
  def body(a_ref, rp_ref, bt_ref, w1_ref, b1_ref, w2_ref, b2_ref,
           w3_ref, b3_ref, wo_ref, bo_ref, out_ref, pool_ref, cnt_ref):
    i = pl.program_id(0)

    @pl.when(i == 0)
    def _init():
      pool_ref[...] = jnp.zeros_like(pool_ref)
      cnt_ref[...] = jnp.zeros_like(cnt_ref)

    h = jnp.maximum(a_ref[0] + a_ref[1] + rp_ref[...], 0.0)
    labels = lax.broadcasted_iota(jnp.int32, (_G, _R), 0)
    onehot = (labels == bt_ref[0]).astype(jnp.float32)
    pool_ref[...] += jnp.dot(onehot, h, preferred_element_type=jnp.float32)
    cnt_ref[:, 0:1] += jnp.sum(onehot, axis=1, keepdims=True)

    @pl.when(i == _NB - 1)
    def _heads():
      pooled = pool_ref[...] / jnp.maximum(cnt_ref[:, 0:1], 1.0)
      cols = []
      for hd in range(_NCLS):
        hc = jnp.maximum(
            jnp.dot(pooled, w1_ref[hd].T,
                    preferred_element_type=jnp.float32) + b1_ref[hd], 0.0)
        hc = jnp.maximum(
            jnp.dot(hc, w2_ref[hd].T,
                    preferred_element_type=jnp.float32) + b2_ref[hd], 0.0)
        hc = jnp.maximum(
            jnp.dot(hc, w3_ref[hd].T,
                    preferred_element_type=jnp.float32) + b3_ref[hd], 0.0)
        o = jnp.dot(hc, wo_ref[hd].reshape(-1, 1),
                    preferred_element_type=jnp.float32) + bo_ref[0, hd]
        cols.append(o)
      out_ref[...] = jnp.concatenate(cols, axis=1)

  full = lambda s: pl.BlockSpec(s, lambda i: tuple(0 for _ in s))
  return pl.pallas_call(
      body,
      grid=(_NB,),
      in_specs=[
          pl.BlockSpec((_NC, _R, 64), lambda i: (0, i, 0)),
          pl.BlockSpec((_R, 64), lambda i: (i, 0)),
          pl.BlockSpec((1, 1, _R), lambda i: (i, 0, 0)),
          full(w1s.shape), full(b1s.shape), full(w2s.shape), full(b2s.shape),
          full(w3s.shape), full(b3s.shape), full(wos.shape), full(bos.shape),
      ],
      out_specs=pl.BlockSpec((_G, _NCLS), lambda i: (0, 0)),
      out_shape=jax.ShapeDtypeStruct((_G, _NCLS), jnp.float32),
      scratch_shapes=[
          pltpu.VMEM((_G, 64), jnp.float32),
          pltpu.VMEM((_G, 128), jnp.float32),
      ],
  )(aggp, r_prev, batch3, w1s, b1s, w2s, b2s, w3s, b3s, wos, bos)


@jax.jit
def kernel(x, edge_index, batch, edge_attr, params):
  src = edge_index[0]
  dst = edge_index[1]
  pad = _EPAD - _E
  src_p = jnp.concatenate([src, jnp.zeros((pad,), jnp.int32)])
  src_p = src_p.reshape(_EPAD // _CHUNK, _CHUNK)
  dst_p = jnp.concatenate([dst, jnp.zeros((pad,), jnp.int32)])
  dst_p = dst_p.reshape(_EPAD // _CHUNK, _CHUNK)
  w_p = jnp.concatenate([edge_attr, jnp.zeros((pad,), jnp.float32)])
  batch3 = batch.reshape(_NB, 1, _R)
  zeros = {c: jnp.zeros((_N, c), jnp.float32) for c in (32, 64)}

  gcn = params['gcn']
  g, r = _proj_first(x, gcn[0]['W_rel'], gcn[0]['b_rel'], gcn[0]['W_root'])
  for li in range(1, len(gcn)):
    cout_prev = g.shape[1]
    aggp = _edge_aggregate(cout_prev)(g, src_p, dst_p, w_p, zeros[cout_prev])
    g, r = _proj_mid(aggp, r, gcn[li]['W_rel'], gcn[li]['b_rel'],
                     gcn[li]['W_root'])
  aggp = _edge_aggregate(64)(g, src_p, dst_p, w_p, zeros[64])

  w1s = jnp.stack([m[0]['W'] for m in params['mlp']])
  b1s = jnp.stack([m[0]['b'] for m in params['mlp']])
  w2s = jnp.stack([m[1]['W'] for m in params['mlp']])
  b2s = jnp.stack([m[1]['b'] for m in params['mlp']])
  w3s = jnp.stack([m[2]['W'] for m in params['mlp']])
  b3s = jnp.stack([m[2]['b'] for m in params['mlp']])
  wos = jnp.stack([o['W'].reshape(-1) for o in params['out']])
  bos = jnp.stack([o['b'].reshape(()) for o in params['out']]).reshape(1, -1)

  return _pool_and_heads(aggp, r, batch3, w1s, b1s, w2s, b2s, w3s, b3s,
                         wos, bos)


# speedup vs baseline: 1.4967x; 1.4967x over previous
import functools

import jax
import jax.numpy as jnp
from jax import lax
from jax.experimental import pallas as pl
from jax.experimental.pallas import tpu as pltpu
from jax.experimental.pallas import tpu_sc as plsc

_N = 10000
_E = 160000
_G = 64
_NCLS = 12
_NC = 2
_NS = 16
_NW = _NC * _NS
_CHUNK = 128
_CPW = 40
_EPAD = _CHUNK * _NW * _CPW
_RPT0 = 632
_RPTL = _N - (_NS - 1) * _RPT0

_R = 2000
_NB = _N // _R


@functools.lru_cache(None)
def _edge_aggregate(C: int):
  mesh = plsc.VectorSubcoreMesh(core_axis_name="c", subcore_axis_name="s")

  @functools.partial(
      pl.kernel,
      mesh=mesh,
      compiler_params=pltpu.CompilerParams(use_tc_tiling_on_sc=False),
      out_type=jax.ShapeDtypeStruct((_NC, _N, C), jnp.float32),
      scratch_types=[
          pltpu.VMEM((_CPW, _CHUNK), jnp.int32),
          pltpu.VMEM((_CPW, _CHUNK), jnp.int32),
          pltpu.VMEM((_CPW * _CHUNK,), jnp.float32),
          pltpu.VMEM((16,), jnp.float32),
          pltpu.VMEM((_CHUNK, C // 2), jnp.int32),
          pltpu.VMEM((_CHUNK, C // 2), jnp.int32),
          pltpu.VMEM((_CHUNK, C // 2), jnp.int32),
          pltpu.VMEM((_CHUNK, C // 2), jnp.int32),
          pltpu.VMEM((_CHUNK, C), jnp.float32),
          pltpu.VMEM((_CHUNK, C), jnp.float32),
          pltpu.VMEM((_CHUNK, C), jnp.float32),
          pltpu.VMEM((_CHUNK, C), jnp.float32),
          pltpu.VMEM_SHARED((_N, C), jnp.float32),
          pltpu.SemaphoreType.DMA,
          pltpu.SemaphoreType.DMA,
          pltpu.SemaphoreType.DMA,
          pltpu.SemaphoreType.DMA,
          pltpu.SemaphoreType.DMA,
          pltpu.SemaphoreType.DMA,
          pltpu.SemaphoreType.DMA,
          pltpu.SemaphoreType.DMA,
          pltpu.SemaphoreType.DMA,
      ],
  )
  def agg_kernel(g_hbm, src_hbm, dst_hbm, w_hbm, s16_hbm,
                 zero_hbm, out_hbm,
                 src_v, dst_v, w_v, s16_v,
                 rows_0, rows_1, rows_2, rows_3,
                 frows_0, frows_1, frows_2, frows_3, acc_sp,
                 sem_g0, sem_g1, sem_g2, sem_g3,
                 sem_s0, sem_s1, sem_s2, sem_s3, sem_ix):
    rows = [rows_0, rows_1, rows_2, rows_3]
    frows = [frows_0, frows_1, frows_2, frows_3]
    sem_g = [sem_g0, sem_g1, sem_g2, sem_g3]
    sem_s = [sem_s0, sem_s1, sem_s2, sem_s3]
    core = lax.axis_index("c")
    sub = lax.axis_index("s")
    wid = sub * _NC + core
    cbase = wid * _CPW
    pltpu.async_copy(src_hbm.at[pl.ds(cbase, _CPW)], src_v, sem_ix)
    pltpu.async_copy(dst_hbm.at[pl.ds(cbase, _CPW)], dst_v, sem_ix)
    pltpu.async_copy(w_hbm.at[pl.ds(cbase * _CHUNK, _CPW * _CHUNK)], w_v,
                     sem_ix)
    pltpu.async_copy(s16_hbm, s16_v, sem_ix)
    start = pl.multiple_of(sub * _RPT0, 8)

    @pl.when(sub < _NS - 1)
    def _clr_main():
      pltpu.sync_copy(zero_hbm.at[pl.ds(start, _RPT0)],
                      acc_sp.at[pl.ds(start, _RPT0)])

    @pl.when(sub == _NS - 1)
    def _clr_tail():
      pltpu.sync_copy(zero_hbm.at[pl.ds(start, _RPTL)],
                      acc_sp.at[pl.ds(start, _RPTL)])

    pltpu.make_async_copy(src_hbm.at[pl.ds(cbase, _CPW)], src_v, sem_ix).wait()
    pltpu.make_async_copy(dst_hbm.at[pl.ds(cbase, _CPW)], dst_v, sem_ix).wait()
    pltpu.make_async_copy(w_hbm.at[pl.ds(cbase * _CHUNK, _CPW * _CHUNK)],
                          w_v, sem_ix).wait()
    pltpu.make_async_copy(s16_hbm, s16_v, sem_ix).wait()
    plsc.subcore_barrier()

    gd = lax.GatherDimensionNumbers(offset_dims=(), collapsed_slice_dims=(0,),
                                    start_index_map=(0,))

    def step(c, b):
      pltpu.make_async_copy(g_hbm.at[src_v.at[c]], rows[b], sem_g[b]).wait()
      bn = (b + 3) % 4
      @pl.when(c + 3 < _CPW)
      def _():
        pltpu.async_copy(g_hbm.at[src_v.at[c + 3]], rows[bn], sem_g[bn])

      @pl.when(c > 0)
      def _():
        pltpu.make_async_copy(frows[bn], acc_sp.at[dst_v.at[c - 1]],
                              sem_s[bn]).wait()

      wbase = pl.multiple_of(c * _CHUNK, _CHUNK)

      svec = s16_v[...]

      def scale_grp(j, carry):
        w16 = w_v[pl.ds(wbase + j * 16, 16)]
        src16 = src_v[c, pl.ds(j * 16, 16)]
        qb = ((src16 >> 4) * 16778) >> 21
        sv = lax.gather(svec, qb.reshape(16, 1), gd, slice_sizes=(1,),
                        mode=lax.GatherScatterMode.PROMISE_IN_BOUNDS)
        ws16 = w16 * sv
        for l in range(16):
          e = j * 16 + l
          wspl = lax.gather(ws16, jnp.full((16, 1), l, jnp.int32), gd,
                            slice_sizes=(1,),
                            mode=lax.GatherScatterMode.PROMISE_IN_BOUNDS)
          for cb in range(C // 32):
            vi = rows[b][e, pl.ds(cb * 16, 16)]
            flo = ((vi << 16) >> 16).astype(jnp.float32)
            fhi = (vi >> 16).astype(jnp.float32)
            frows[b][e, pl.ds(cb * 16, 16)] = flo * wspl
            frows[b][e, pl.ds(cb * 16 + C // 2, 16)] = fhi * wspl
        return carry

      lax.fori_loop(0, _CHUNK // 16, scale_grp, 0, unroll=True)
      pltpu.async_copy(frows[b], acc_sp.at[dst_v.at[c]], sem_s[b], add=True)

    for b in range(3):
      pltpu.async_copy(g_hbm.at[src_v.at[b]], rows[b], sem_g[b])

    def run_quad(q, carry):
      for b in range(4):
        step(4 * q + b, b)
      return carry

    lax.fori_loop(0, _CPW // 4, run_quad, 0)
    pltpu.make_async_copy(frows[3], acc_sp.at[dst_v.at[_CPW - 1]],
                          sem_s[3]).wait()
    plsc.subcore_barrier()

    @pl.when(sub < _NS - 1)
    def _wb_main():
      pltpu.sync_copy(acc_sp.at[pl.ds(start, _RPT0)],
                      out_hbm.at[core, pl.ds(start, _RPT0)])

    @pl.when(sub == _NS - 1)
    def _wb_tail():
      pltpu.sync_copy(acc_sp.at[pl.ds(start, _RPTL)],
                      out_hbm.at[core, pl.ds(start, _RPTL)])

  return agg_kernel


def _proj_first(x, w_rel, b_rel, w_root):
  cin = x.shape[1]
  cout = w_rel.shape[0]
  wcat = jnp.concatenate([w_rel, w_root], axis=0)

  def body(x_ref, w_ref, b_ref, g_ref, r_ref, s_ref):
    h = x_ref[...]
    gr = jnp.dot(h, w_ref[...].T, preferred_element_type=jnp.float32)
    g = gr[:, :cout]
    smax = jnp.maximum(jnp.max(jnp.abs(g)), 1e-30)
    s = smax * (1.0 / 32000.0)
    q = jnp.round(g * (32000.0 / smax)).astype(jnp.int32)
    qlo = q[:, :cout // 2] & 0xFFFF
    qhi = q[:, cout // 2:] << 16
    g_ref[...] = qhi | qlo
    r_ref[...] = gr[:, cout:] + b_ref[...]
    s_ref[...] = jnp.full((1, 8, 128), s, jnp.float32)

  return pl.pallas_call(
      body,
      grid=(_NB,),
      in_specs=[
          pl.BlockSpec((_R, cin), lambda i: (i, 0)),
          pl.BlockSpec((2 * cout, cin), lambda i: (0, 0)),
          pl.BlockSpec((1, cout), lambda i: (0, 0)),
      ],
      out_specs=[
          pl.BlockSpec((_R, cout // 2), lambda i: (i, 0)),
          pl.BlockSpec((_R, cout), lambda i: (i, 0)),
          pl.BlockSpec((1, 8, 128), lambda i: (i, 0, 0)),
      ],
      out_shape=[
          jax.ShapeDtypeStruct((_N, cout // 2), jnp.int32),
          jax.ShapeDtypeStruct((_N, cout), jnp.float32),
          jax.ShapeDtypeStruct((_NB, 8, 128), jnp.float32),
      ],
  )(x, wcat, b_rel.reshape(1, -1))


def _proj_mid(aggp, r_prev, w_rel, b_rel, w_root):
  cin = r_prev.shape[1]
  cout = w_rel.shape[0]
  wcat = jnp.concatenate([w_rel, w_root], axis=0)

  def body(a_ref, rp_ref, w_ref, b_ref, g_ref, r_ref, s_ref):
    h = jnp.maximum(a_ref[0] + a_ref[1] + rp_ref[...], 0.0)
    gr = jnp.dot(h, w_ref[...].T, preferred_element_type=jnp.float32)
    g = gr[:, :cout]
    smax = jnp.maximum(jnp.max(jnp.abs(g)), 1e-30)
    s = smax * (1.0 / 32000.0)
    q = jnp.round(g * (32000.0 / smax)).astype(jnp.int32)
    qlo = q[:, :cout // 2] & 0xFFFF
    qhi = q[:, cout // 2:] << 16
    g_ref[...] = qhi | qlo
    r_ref[...] = gr[:, cout:] + b_ref[...]
    s_ref[...] = jnp.full((1, 8, 128), s, jnp.float32)

  return pl.pallas_call(
      body,
      grid=(_NB,),
      in_specs=[
          pl.BlockSpec((_NC, _R, cin), lambda i: (0, i, 0)),
          pl.BlockSpec((_R, cin), lambda i: (i, 0)),
          pl.BlockSpec((2 * cout, cin), lambda i: (0, 0)),
          pl.BlockSpec((1, cout), lambda i: (0, 0)),
      ],
      out_specs=[
          pl.BlockSpec((_R, cout // 2), lambda i: (i, 0)),
          pl.BlockSpec((_R, cout), lambda i: (i, 0)),
          pl.BlockSpec((1, 8, 128), lambda i: (i, 0, 0)),
      ],
      out_shape=[
          jax.ShapeDtypeStruct((_N, cout // 2), jnp.int32),
          jax.ShapeDtypeStruct((_N, cout), jnp.float32),
          jax.ShapeDtypeStruct((_NB, 8, 128), jnp.float32),
      ],
  )(aggp, r_prev, wcat, b_rel.reshape(1, -1))


def _pool_and_heads(aggp, r_prev, batch3, w1s, b1s, w2s, b2s, w3s, b3s,
                    wos, bos):
  def body(a_ref, rp_ref, bt_ref, w1_ref, b1_ref, w2_ref, b2_ref,
           w3_ref, b3_ref, wo_ref, bo_ref, out_ref, pool_ref, cnt_ref):
    i = pl.program_id(0)

    @pl.when(i == 0)
    def _init():
      pool_ref[...] = jnp.zeros_like(pool_ref)
      cnt_ref[...] = jnp.zeros_like(cnt_ref)

    h = jnp.maximum(a_ref[0] + a_ref[1] + rp_ref[...], 0.0)
    labels = lax.broadcasted_iota(jnp.int32, (_G, _R), 0)
    onehot = (labels == bt_ref[0]).astype(jnp.float32)
    pool_ref[...] += jnp.dot(onehot, h, preferred_element_type=jnp.float32)
    cnt_ref[:, 0:1] += jnp.sum(onehot, axis=1, keepdims=True)

    @pl.when(i == _NB - 1)
    def _heads():
      pooled = pool_ref[...] / jnp.maximum(cnt_ref[:, 0:1], 1.0)
      cols = []
      for hd in range(_NCLS):
        hc = jnp.maximum(
            jnp.dot(pooled, w1_ref[hd].T,
                    preferred_element_type=jnp.float32) + b1_ref[hd], 0.0)
        hc = jnp.maximum(
            jnp.dot(hc, w2_ref[hd].T,
                    preferred_element_type=jnp.float32) + b2_ref[hd], 0.0)
        hc = jnp.maximum(
            jnp.dot(hc, w3_ref[hd].T,
                    preferred_element_type=jnp.float32) + b3_ref[hd], 0.0)
        o = jnp.dot(hc, wo_ref[hd].reshape(-1, 1),
                    preferred_element_type=jnp.float32) + bo_ref[0, hd]
        cols.append(o)
      out_ref[...] = jnp.concatenate(cols, axis=1)

  full = lambda s: pl.BlockSpec(s, lambda i: tuple(0 for _ in s))
  return pl.pallas_call(
      body,
      grid=(_NB,),
      in_specs=[
          pl.BlockSpec((_NC, _R, 64), lambda i: (0, i, 0)),
          pl.BlockSpec((_R, 64), lambda i: (i, 0)),
          pl.BlockSpec((1, 1, _R), lambda i: (i, 0, 0)),
          full(w1s.shape), full(b1s.shape), full(w2s.shape), full(b2s.shape),
          full(w3s.shape), full(b3s.shape), full(wos.shape), full(bos.shape),
      ],
      out_specs=pl.BlockSpec((_G, _NCLS), lambda i: (0, 0)),
      out_shape=jax.ShapeDtypeStruct((_G, _NCLS), jnp.float32),
      scratch_shapes=[
          pltpu.VMEM((_G, 64), jnp.float32),
          pltpu.VMEM((_G, 128), jnp.float32),
      ],
  )(aggp, r_prev, batch3, w1s, b1s, w2s, b2s, w3s, b3s, wos, bos)


@jax.jit
def kernel(x, edge_index, batch, edge_attr, params):
  src = edge_index[0]
  dst = edge_index[1]
  pad = _EPAD - _E
  src_p = jnp.concatenate([src, jnp.zeros((pad,), jnp.int32)])
  src_p = src_p.reshape(_EPAD // _CHUNK, _CHUNK)
  dst_p = jnp.concatenate([dst, jnp.zeros((pad,), jnp.int32)])
  dst_p = dst_p.reshape(_EPAD // _CHUNK, _CHUNK)
  w_p = jnp.concatenate([edge_attr, jnp.zeros((pad,), jnp.float32)])
  batch3 = batch.reshape(_NB, 1, _R)
  zeros = {c: jnp.zeros((_N, c), jnp.float32) for c in (32, 64)}

  gcn = params['gcn']
  g, r, s_out = _proj_first(x, gcn[0]['W_rel'], gcn[0]['b_rel'],
                            gcn[0]['W_root'])
  for li in range(1, len(gcn)):
    cout_prev = 2 * g.shape[1]
    s16 = jnp.pad(s_out[:, 0, 0], (0, 16 - _NB))
    aggp = _edge_aggregate(cout_prev)(g, src_p, dst_p, w_p, s16,
                                      zeros[cout_prev])
    g, r, s_out = _proj_mid(aggp, r, gcn[li]['W_rel'], gcn[li]['b_rel'],
                            gcn[li]['W_root'])
  s16 = jnp.pad(s_out[:, 0, 0], (0, 16 - _NB))
  aggp = _edge_aggregate(64)(g, src_p, dst_p, w_p, s16, zeros[64])

  w1s = jnp.stack([m[0]['W'] for m in params['mlp']])
  b1s = jnp.stack([m[0]['b'] for m in params['mlp']])
  w2s = jnp.stack([m[1]['W'] for m in params['mlp']])
  b2s = jnp.stack([m[1]['b'] for m in params['mlp']])
  w3s = jnp.stack([m[2]['W'] for m in params['mlp']])
  b3s = jnp.stack([m[2]['b'] for m in params['mlp']])
  wos = jnp.stack([o['W'].reshape(-1) for o in params['out']])
  bos = jnp.stack([o['b'].reshape(()) for o in params['out']]).reshape(1, -1)

  return _pool_and_heads(aggp, r, batch3, w1s, b1s, w2s, b2s, w3s, b3s,
                         wos, bos)


# speedup vs baseline: 1.6600x; 1.1091x over previous
import functools

import jax
import jax.numpy as jnp
from jax import lax
from jax.experimental import pallas as pl
from jax.experimental.pallas import tpu as pltpu
from jax.experimental.pallas import tpu_sc as plsc

_N = 10000
_E = 160000
_G = 64
_NCLS = 12
_NC = 2
_NS = 16
_NW = _NC * _NS
_CHUNK = 128
_CPW = 40
_EPAD = _CHUNK * _NW * _CPW
_RPT0 = 632
_RPTL = _N - (_NS - 1) * _RPT0

_R = 2000
_NB = _N // _R


_D = 5


@functools.lru_cache(None)
def _edge_aggregate(C: int):
  mesh = plsc.VectorSubcoreMesh(core_axis_name="c", subcore_axis_name="s")

  @functools.partial(
      pl.kernel,
      mesh=mesh,
      compiler_params=pltpu.CompilerParams(use_tc_tiling_on_sc=False),
      out_type=jax.ShapeDtypeStruct((_NC, _N, C), jnp.float32),
      scratch_types=(
          [
              pltpu.VMEM((_CPW, _CHUNK), jnp.int32),
              pltpu.VMEM((_CPW, _CHUNK), jnp.int32),
              pltpu.VMEM((_CPW * _CHUNK,), jnp.float32),
              pltpu.VMEM((16,), jnp.float32),
          ]
          + [pltpu.VMEM((_CHUNK, C // 2), jnp.int32) for _ in range(_D)]
          + [pltpu.VMEM((_CHUNK, C), jnp.float32) for _ in range(_D)]
          + [pltpu.VMEM_SHARED((_N, C), jnp.float32)]
          + [pltpu.SemaphoreType.DMA for _ in range(2 * _D + 1)]
      ),
  )
  def agg_kernel(g_hbm, src_hbm, dst_hbm, w_hbm, s16_hbm, out_hbm,
                 src_v, dst_v, w_v, s16_v, *rest):
    rows = list(rest[:_D])
    frows = list(rest[_D:2 * _D])
    acc_sp = rest[2 * _D]
    sem_g = list(rest[2 * _D + 1:3 * _D + 1])
    sem_s = list(rest[3 * _D + 1:4 * _D + 1])
    sem_ix = rest[4 * _D + 1]
    core = lax.axis_index("c")
    sub = lax.axis_index("s")
    wid = sub * _NC + core
    cbase = wid * _CPW
    pltpu.async_copy(src_hbm.at[pl.ds(cbase, _CPW)], src_v, sem_ix)
    pltpu.async_copy(dst_hbm.at[pl.ds(cbase, _CPW)], dst_v, sem_ix)
    pltpu.async_copy(w_hbm.at[pl.ds(cbase * _CHUNK, _CPW * _CHUNK)], w_v,
                     sem_ix)
    pltpu.async_copy(s16_hbm, s16_v, sem_ix)
    pltpu.make_async_copy(src_hbm.at[pl.ds(cbase, _CPW)], src_v, sem_ix).wait()

    for b in range(_D - 1):
      pltpu.async_copy(g_hbm.at[src_v.at[b]], rows[b], sem_g[b])

    zsrc = frows[0]

    def zfill(i, carry):
      for cb in range(C // 16):
        zsrc[i, pl.ds(cb * 16, 16)] = jnp.zeros((16,), jnp.float32)
      return carry

    lax.fori_loop(0, _CHUNK, zfill, 0)
    start = pl.multiple_of(sub * _RPT0, 8)
    for k in range(4):
      pltpu.sync_copy(zsrc, acc_sp.at[pl.ds(start + k * _CHUNK, _CHUNK)])

    @pl.when(sub < _NS - 1)
    def _clr_main():
      pltpu.sync_copy(zsrc.at[pl.ds(0, _RPT0 - 4 * _CHUNK)],
                      acc_sp.at[pl.ds(start + 4 * _CHUNK,
                                      _RPT0 - 4 * _CHUNK)])

    @pl.when(sub == _NS - 1)
    def _clr_tail():
      pltpu.sync_copy(zsrc.at[pl.ds(0, _RPTL - 4 * _CHUNK)],
                      acc_sp.at[pl.ds(start + 4 * _CHUNK,
                                      _RPTL - 4 * _CHUNK)])

    pltpu.make_async_copy(dst_hbm.at[pl.ds(cbase, _CPW)], dst_v, sem_ix).wait()
    pltpu.make_async_copy(w_hbm.at[pl.ds(cbase * _CHUNK, _CPW * _CHUNK)],
                          w_v, sem_ix).wait()
    pltpu.make_async_copy(s16_hbm, s16_v, sem_ix).wait()
    plsc.subcore_barrier()

    gd = lax.GatherDimensionNumbers(offset_dims=(), collapsed_slice_dims=(0,),
                                    start_index_map=(0,))

    def step(c, b):
      pltpu.make_async_copy(g_hbm.at[src_v.at[c]], rows[b], sem_g[b]).wait()
      bn = (b + _D - 1) % _D
      @pl.when(c + _D - 1 < _CPW)
      def _():
        pltpu.async_copy(g_hbm.at[src_v.at[c + _D - 1]], rows[bn], sem_g[bn])

      @pl.when(c > 0)
      def _():
        pltpu.make_async_copy(frows[bn], acc_sp.at[dst_v.at[c - 1]],
                              sem_s[bn]).wait()

      wbase = pl.multiple_of(c * _CHUNK, _CHUNK)
      svec = s16_v[...]

      def scale_grp(j, carry):
        w16 = w_v[pl.ds(wbase + j * 16, 16)]
        src16 = src_v[c, pl.ds(j * 16, 16)]
        qb = ((src16 >> 4) * 16778) >> 21
        sv = lax.gather(svec, qb.reshape(16, 1), gd, slice_sizes=(1,),
                        mode=lax.GatherScatterMode.PROMISE_IN_BOUNDS)
        ws16 = w16 * sv
        for l in range(16):
          e = j * 16 + l
          wspl = lax.gather(ws16, jnp.full((16, 1), l, jnp.int32), gd,
                            slice_sizes=(1,),
                            mode=lax.GatherScatterMode.PROMISE_IN_BOUNDS)
          for cb in range(C // 32):
            vi = rows[b][e, pl.ds(cb * 16, 16)]
            flo = ((vi << 16) >> 16).astype(jnp.float32)
            fhi = (vi >> 16).astype(jnp.float32)
            frows[b][e, pl.ds(cb * 16, 16)] = flo * wspl
            frows[b][e, pl.ds(cb * 16 + C // 2, 16)] = fhi * wspl
        return carry

      lax.fori_loop(0, _CHUNK // 16, scale_grp, 0, unroll=True)
      pltpu.async_copy(frows[b], acc_sp.at[dst_v.at[c]], sem_s[b], add=True)

    def run_group(q, carry):
      for b in range(_D):
        step(_D * q + b, b)
      return carry

    lax.fori_loop(0, _CPW // _D, run_group, 0)
    pltpu.make_async_copy(frows[(_CPW - 1) % _D],
                          acc_sp.at[dst_v.at[_CPW - 1]],
                          sem_s[(_CPW - 1) % _D]).wait()
    plsc.subcore_barrier()

    @pl.when(sub < _NS - 1)
    def _wb_main():
      pltpu.sync_copy(acc_sp.at[pl.ds(start, _RPT0)],
                      out_hbm.at[core, pl.ds(start, _RPT0)])

    @pl.when(sub == _NS - 1)
    def _wb_tail():
      pltpu.sync_copy(acc_sp.at[pl.ds(start, _RPTL)],
                      out_hbm.at[core, pl.ds(start, _RPTL)])

  return agg_kernel


def _proj_first(x, w_rel, b_rel, w_root):
  cin = x.shape[1]
  cout = w_rel.shape[0]
  wcat = jnp.concatenate([w_rel, w_root], axis=0)

  def body(x_ref, w_ref, b_ref, g_ref, r_ref, s_ref):
    h = x_ref[...]
    gr = jnp.dot(h, w_ref[...].T, preferred_element_type=jnp.float32)
    g = gr[:, :cout]
    smax = jnp.maximum(jnp.max(jnp.abs(g)), 1e-30)
    s = smax * (1.0 / 32000.0)
    q = jnp.round(g * (32000.0 / smax)).astype(jnp.int32)
    qlo = q[:, :cout // 2] & 0xFFFF
    qhi = q[:, cout // 2:] << 16
    g_ref[...] = qhi | qlo
    r_ref[...] = gr[:, cout:] + b_ref[...]
    s_ref[...] = jnp.full((1, 8, 128), s, jnp.float32)

  return pl.pallas_call(
      body,
      grid=(_NB,),
      in_specs=[
          pl.BlockSpec((_R, cin), lambda i: (i, 0)),
          pl.BlockSpec((2 * cout, cin), lambda i: (0, 0)),
          pl.BlockSpec((1, cout), lambda i: (0, 0)),
      ],
      out_specs=[
          pl.BlockSpec((_R, cout // 2), lambda i: (i, 0)),
          pl.BlockSpec((_R, cout), lambda i: (i, 0)),
          pl.BlockSpec((1, 8, 128), lambda i: (i, 0, 0)),
      ],
      out_shape=[
          jax.ShapeDtypeStruct((_N, cout // 2), jnp.int32),
          jax.ShapeDtypeStruct((_N, cout), jnp.float32),
          jax.ShapeDtypeStruct((_NB, 8, 128), jnp.float32),
      ],
  )(x, wcat, b_rel.reshape(1, -1))


def _proj_mid(aggp, r_prev, w_rel, b_rel, w_root):
  cin = r_prev.shape[1]
  cout = w_rel.shape[0]
  wcat = jnp.concatenate([w_rel, w_root], axis=0)

  def body(a_ref, rp_ref, w_ref, b_ref, g_ref, r_ref, s_ref):
    h = jnp.maximum(a_ref[0] + a_ref[1] + rp_ref[...], 0.0)
    gr = jnp.dot(h, w_ref[...].T, preferred_element_type=jnp.float32)
    g = gr[:, :cout]
    smax = jnp.maximum(jnp.max(jnp.abs(g)), 1e-30)
    s = smax * (1.0 / 32000.0)
    q = jnp.round(g * (32000.0 / smax)).astype(jnp.int32)
    qlo = q[:, :cout // 2] & 0xFFFF
    qhi = q[:, cout // 2:] << 16
    g_ref[...] = qhi | qlo
    r_ref[...] = gr[:, cout:] + b_ref[...]
    s_ref[...] = jnp.full((1, 8, 128), s, jnp.float32)

  return pl.pallas_call(
      body,
      grid=(_NB,),
      in_specs=[
          pl.BlockSpec((_NC, _R, cin), lambda i: (0, i, 0)),
          pl.BlockSpec((_R, cin), lambda i: (i, 0)),
          pl.BlockSpec((2 * cout, cin), lambda i: (0, 0)),
          pl.BlockSpec((1, cout), lambda i: (0, 0)),
      ],
      out_specs=[
          pl.BlockSpec((_R, cout // 2), lambda i: (i, 0)),
          pl.BlockSpec((_R, cout), lambda i: (i, 0)),
          pl.BlockSpec((1, 8, 128), lambda i: (i, 0, 0)),
      ],
      out_shape=[
          jax.ShapeDtypeStruct((_N, cout // 2), jnp.int32),
          jax.ShapeDtypeStruct((_N, cout), jnp.float32),
          jax.ShapeDtypeStruct((_NB, 8, 128), jnp.float32),
      ],
  )(aggp, r_prev, wcat, b_rel.reshape(1, -1))


def _pool_and_heads(aggp, r_prev, batch3, w1s, b1s, w2s, b2s, w3s, b3s,
                    wos, bos):
  def body(a_ref, rp_ref, bt_ref, w1_ref, b1_ref, w2_ref, b2_ref,
           w3_ref, b3_ref, wo_ref, bo_ref, out_ref, pool_ref, cnt_ref):
    i = pl.program_id(0)

    @pl.when(i == 0)
    def _init():
      pool_ref[...] = jnp.zeros_like(pool_ref)
      cnt_ref[...] = jnp.zeros_like(cnt_ref)

    h = jnp.maximum(a_ref[0] + a_ref[1] + rp_ref[...], 0.0)
    labels = lax.broadcasted_iota(jnp.int32, (_G, _R), 0)
    onehot = (labels == bt_ref[0]).astype(jnp.float32)
    pool_ref[...] += jnp.dot(onehot, h, preferred_element_type=jnp.float32)
    cnt_ref[:, 0:1] += jnp.sum(onehot, axis=1, keepdims=True)

    @pl.when(i == _NB - 1)
    def _heads():
      pooled = pool_ref[...] / jnp.maximum(cnt_ref[:, 0:1], 1.0)
      cols = []
      for hd in range(_NCLS):
        hc = jnp.maximum(
            jnp.dot(pooled, w1_ref[hd].T,
                    preferred_element_type=jnp.float32) + b1_ref[hd], 0.0)
        hc = jnp.maximum(
            jnp.dot(hc, w2_ref[hd].T,
                    preferred_element_type=jnp.float32) + b2_ref[hd], 0.0)
        hc = jnp.maximum(
            jnp.dot(hc, w3_ref[hd].T,
                    preferred_element_type=jnp.float32) + b3_ref[hd], 0.0)
        o = jnp.dot(hc, wo_ref[hd].reshape(-1, 1),
                    preferred_element_type=jnp.float32) + bo_ref[0, hd]
        cols.append(o)
      out_ref[...] = jnp.concatenate(cols, axis=1)

  full = lambda s: pl.BlockSpec(s, lambda i: tuple(0 for _ in s))
  return pl.pallas_call(
      body,
      grid=(_NB,),
      in_specs=[
          pl.BlockSpec((_NC, _R, 64), lambda i: (0, i, 0)),
          pl.BlockSpec((_R, 64), lambda i: (i, 0)),
          pl.BlockSpec((1, 1, _R), lambda i: (i, 0, 0)),
          full(w1s.shape), full(b1s.shape), full(w2s.shape), full(b2s.shape),
          full(w3s.shape), full(b3s.shape), full(wos.shape), full(bos.shape),
      ],
      out_specs=pl.BlockSpec((_G, _NCLS), lambda i: (0, 0)),
      out_shape=jax.ShapeDtypeStruct((_G, _NCLS), jnp.float32),
      scratch_shapes=[
          pltpu.VMEM((_G, 64), jnp.float32),
          pltpu.VMEM((_G, 128), jnp.float32),
      ],
  )(aggp, r_prev, batch3, w1s, b1s, w2s, b2s, w3s, b3s, wos, bos)


@jax.jit
def kernel(x, edge_index, batch, edge_attr, params):
  src = edge_index[0]
  dst = edge_index[1]
  pad = _EPAD - _E
  src_p = jnp.concatenate([src, jnp.zeros((pad,), jnp.int32)])
  src_p = src_p.reshape(_EPAD // _CHUNK, _CHUNK)
  dst_p = jnp.concatenate([dst, jnp.zeros((pad,), jnp.int32)])
  dst_p = dst_p.reshape(_EPAD // _CHUNK, _CHUNK)
  w_p = jnp.concatenate([edge_attr, jnp.zeros((pad,), jnp.float32)])
  batch3 = batch.reshape(_NB, 1, _R)

  gcn = params['gcn']
  g, r, s_out = _proj_first(x, gcn[0]['W_rel'], gcn[0]['b_rel'],
                            gcn[0]['W_root'])
  for li in range(1, len(gcn)):
    cout_prev = 2 * g.shape[1]
    s16 = jnp.pad(s_out[:, 0, 0], (0, 16 - _NB))
    aggp = _edge_aggregate(cout_prev)(g, src_p, dst_p, w_p, s16)
    g, r, s_out = _proj_mid(aggp, r, gcn[li]['W_rel'], gcn[li]['b_rel'],
                            gcn[li]['W_root'])
  s16 = jnp.pad(s_out[:, 0, 0], (0, 16 - _NB))
  aggp = _edge_aggregate(64)(g, src_p, dst_p, w_p, s16)

  w1s = jnp.stack([m[0]['W'] for m in params['mlp']])
  b1s = jnp.stack([m[0]['b'] for m in params['mlp']])
  w2s = jnp.stack([m[1]['W'] for m in params['mlp']])
  b2s = jnp.stack([m[1]['b'] for m in params['mlp']])
  w3s = jnp.stack([m[2]['W'] for m in params['mlp']])
  b3s = jnp.stack([m[2]['b'] for m in params['mlp']])
  wos = jnp.stack([o['W'].reshape(-1) for o in params['out']])
  bos = jnp.stack([o['b'].reshape(()) for o in params['out']]).reshape(1, -1)

  return _pool_and_heads(aggp, r, batch3, w1s, b1s, w2s, b2s, w3s, b3s,
                         wos, bos)


# speedup vs baseline: 1.6632x; 1.0019x over previous
import functools

import jax
import jax.numpy as jnp
from jax import lax
from jax.experimental import pallas as pl
from jax.experimental.pallas import tpu as pltpu
from jax.experimental.pallas import tpu_sc as plsc

_N = 10000
_E = 160000
_G = 64
_NCLS = 12
_NC = 2
_NS = 16
_NW = _NC * _NS
_CHUNK = 128
_CPW = 40
_EPAD = _CHUNK * _NW * _CPW
_RPT0 = 632
_RPTL = _N - (_NS - 1) * _RPT0

_R = 2000
_NB = _N // _R


_D = 5


@functools.lru_cache(None)
def _edge_aggregate(C: int):
  mesh = plsc.VectorSubcoreMesh(core_axis_name="c", subcore_axis_name="s")

  @functools.partial(
      pl.kernel,
      mesh=mesh,
      compiler_params=pltpu.CompilerParams(use_tc_tiling_on_sc=False),
      out_type=jax.ShapeDtypeStruct((_NC, _N, C), jnp.float32),
      scratch_types=(
          [
              pltpu.VMEM((_CPW, _CHUNK), jnp.int32),
              pltpu.VMEM((_CPW, _CHUNK), jnp.int32),
              pltpu.VMEM((_CPW * _CHUNK,), jnp.float32),
              pltpu.VMEM((16,), jnp.float32),
          ]
          + [pltpu.VMEM((_CHUNK, C // 2), jnp.int32) for _ in range(_D)]
          + [pltpu.VMEM((_CHUNK, C), jnp.float32) for _ in range(_D)]
          + [pltpu.VMEM_SHARED((_N, C), jnp.float32)]
          + [pltpu.SemaphoreType.DMA for _ in range(2 * _D + 1)]
      ),
  )
  def agg_kernel(g_hbm, src_hbm, dst_hbm, w_hbm, s16_hbm, out_hbm,
                 src_v, dst_v, w_v, s16_v, *rest):
    rows = list(rest[:_D])
    frows = list(rest[_D:2 * _D])
    acc_sp = rest[2 * _D]
    sem_g = list(rest[2 * _D + 1:3 * _D + 1])
    sem_s = list(rest[3 * _D + 1:4 * _D + 1])
    sem_ix = rest[4 * _D + 1]
    core = lax.axis_index("c")
    sub = lax.axis_index("s")
    wid = core * _NS + sub
    cbase = wid * _CPW
    pltpu.async_copy(src_hbm.at[pl.ds(cbase, _CPW)], src_v, sem_ix)
    pltpu.async_copy(dst_hbm.at[pl.ds(cbase, _CPW)], dst_v, sem_ix)
    pltpu.async_copy(w_hbm.at[pl.ds(cbase * _CHUNK, _CPW * _CHUNK)], w_v,
                     sem_ix)
    pltpu.async_copy(s16_hbm, s16_v, sem_ix)
    pltpu.make_async_copy(src_hbm.at[pl.ds(cbase, _CPW)], src_v, sem_ix).wait()

    for b in range(_D - 1):
      pltpu.async_copy(g_hbm.at[src_v.at[b]], rows[b], sem_g[b])

    zsrc = frows[0]

    def zfill(i, carry):
      for cb in range(C // 16):
        zsrc[i, pl.ds(cb * 16, 16)] = jnp.zeros((16,), jnp.float32)
      return carry

    lax.fori_loop(0, _CHUNK, zfill, 0)
    start = pl.multiple_of(sub * _RPT0, 8)
    for k in range(4):
      pltpu.sync_copy(zsrc, acc_sp.at[pl.ds(start + k * _CHUNK, _CHUNK)])

    @pl.when(sub < _NS - 1)
    def _clr_main():
      pltpu.sync_copy(zsrc.at[pl.ds(0, _RPT0 - 4 * _CHUNK)],
                      acc_sp.at[pl.ds(start + 4 * _CHUNK,
                                      _RPT0 - 4 * _CHUNK)])

    @pl.when(sub == _NS - 1)
    def _clr_tail():
      pltpu.sync_copy(zsrc.at[pl.ds(0, _RPTL - 4 * _CHUNK)],
                      acc_sp.at[pl.ds(start + 4 * _CHUNK,
                                      _RPTL - 4 * _CHUNK)])

    pltpu.make_async_copy(dst_hbm.at[pl.ds(cbase, _CPW)], dst_v, sem_ix).wait()
    pltpu.make_async_copy(w_hbm.at[pl.ds(cbase * _CHUNK, _CPW * _CHUNK)],
                          w_v, sem_ix).wait()
    pltpu.make_async_copy(s16_hbm, s16_v, sem_ix).wait()
    plsc.subcore_barrier()

    gd = lax.GatherDimensionNumbers(offset_dims=(), collapsed_slice_dims=(0,),
                                    start_index_map=(0,))

    def step(c, b):
      pltpu.make_async_copy(g_hbm.at[src_v.at[c]], rows[b], sem_g[b]).wait()
      bn = (b + _D - 1) % _D
      @pl.when(c + _D - 1 < _CPW)
      def _():
        pltpu.async_copy(g_hbm.at[src_v.at[c + _D - 1]], rows[bn], sem_g[bn])

      @pl.when(c > 0)
      def _():
        pltpu.make_async_copy(frows[bn], acc_sp.at[dst_v.at[c - 1]],
                              sem_s[bn]).wait()

      wbase = pl.multiple_of(c * _CHUNK, _CHUNK)
      svec = s16_v[...]

      def scale_grp(j, carry):
        w16 = w_v[pl.ds(wbase + j * 16, 16)]
        src16 = src_v[c, pl.ds(j * 16, 16)]
        qb = ((src16 >> 4) * 16778) >> 21
        sv = lax.gather(svec, qb.reshape(16, 1), gd, slice_sizes=(1,),
                        mode=lax.GatherScatterMode.PROMISE_IN_BOUNDS)
        ws16 = w16 * sv
        for l in range(16):
          e = j * 16 + l
          wspl = lax.gather(ws16, jnp.full((16, 1), l, jnp.int32), gd,
                            slice_sizes=(1,),
                            mode=lax.GatherScatterMode.PROMISE_IN_BOUNDS)
          for cb in range(C // 32):
            vi = rows[b][e, pl.ds(cb * 16, 16)]
            flo = ((vi << 16) >> 16).astype(jnp.float32)
            fhi = (vi >> 16).astype(jnp.float32)
            frows[b][e, pl.ds(cb * 16, 16)] = flo * wspl
            frows[b][e, pl.ds(cb * 16 + C // 2, 16)] = fhi * wspl
        return carry

      lax.fori_loop(0, _CHUNK // 16, scale_grp, 0, unroll=True)
      pltpu.async_copy(frows[b], acc_sp.at[dst_v.at[c]], sem_s[b], add=True)

    def run_group(q, carry):
      for b in range(_D):
        step(_D * q + b, b)
      return carry

    lax.fori_loop(0, _CPW // _D, run_group, 0)
    pltpu.make_async_copy(frows[(_CPW - 1) % _D],
                          acc_sp.at[dst_v.at[_CPW - 1]],
                          sem_s[(_CPW - 1) % _D]).wait()
    plsc.subcore_barrier()

    @pl.when(sub < _NS - 1)
    def _wb_main():
      pltpu.sync_copy(acc_sp.at[pl.ds(start, _RPT0)],
                      out_hbm.at[core, pl.ds(start, _RPT0)])

    @pl.when(sub == _NS - 1)
    def _wb_tail():
      pltpu.sync_copy(acc_sp.at[pl.ds(start, _RPTL)],
                      out_hbm.at[core, pl.ds(start, _RPTL)])

  return agg_kernel


def _proj_first(x, w_rel, b_rel, w_root):
  cin = x.shape[1]
  cout = w_rel.shape[0]
  wcat = jnp.concatenate([w_rel, w_root], axis=0)

  def body(x_ref, w_ref, b_ref, g_ref, r_ref, s_ref):
    h = x_ref[...]
    gr = jnp.dot(h, w_ref[...].T, preferred_element_type=jnp.float32)
    g = gr[:, :cout]
    smax = jnp.maximum(jnp.max(jnp.abs(g)), 1e-30)
    s = smax * (1.0 / 32000.0)
    q = jnp.round(g * (32000.0 / smax)).astype(jnp.int32)
    qlo = q[:, :cout // 2] & 0xFFFF
    qhi = q[:, cout // 2:] << 16
    g_ref[...] = qhi | qlo
    r_ref[...] = gr[:, cout:] + b_ref[...]
    s_ref[...] = jnp.full((1, 8, 128), s, jnp.float32)

  return pl.pallas_call(
      body,
      grid=(_NB,),
      in_specs=[
          pl.BlockSpec((_R, cin), lambda i: (i, 0)),
          pl.BlockSpec((2 * cout, cin), lambda i: (0, 0)),
          pl.BlockSpec((1, cout), lambda i: (0, 0)),
      ],
      out_specs=[
          pl.BlockSpec((_R, cout // 2), lambda i: (i, 0)),
          pl.BlockSpec((_R, cout), lambda i: (i, 0)),
          pl.BlockSpec((1, 8, 128), lambda i: (i, 0, 0)),
      ],
      out_shape=[
          jax.ShapeDtypeStruct((_N, cout // 2), jnp.int32),
          jax.ShapeDtypeStruct((_N, cout), jnp.float32),
          jax.ShapeDtypeStruct((_NB, 8, 128), jnp.float32),
      ],
  )(x, wcat, b_rel.reshape(1, -1))


def _proj_mid(aggp, r_prev, w_rel, b_rel, w_root):
  cin = r_prev.shape[1]
  cout = w_rel.shape[0]
  wcat = jnp.concatenate([w_rel, w_root], axis=0)

  def body(a_ref, rp_ref, w_ref, b_ref, g_ref, r_ref, s_ref):
    h = jnp.maximum(a_ref[0] + a_ref[1] + rp_ref[...], 0.0)
    gr = jnp.dot(h, w_ref[...].T, preferred_element_type=jnp.float32)
    g = gr[:, :cout]
    smax = jnp.maximum(jnp.max(jnp.abs(g)), 1e-30)
    s = smax * (1.0 / 32000.0)
    q = jnp.round(g * (32000.0 / smax)).astype(jnp.int32)
    qlo = q[:, :cout // 2] & 0xFFFF
    qhi = q[:, cout // 2:] << 16
    g_ref[...] = qhi | qlo
    r_ref[...] = gr[:, cout:] + b_ref[...]
    s_ref[...] = jnp.full((1, 8, 128), s, jnp.float32)

  return pl.pallas_call(
      body,
      grid=(_NB,),
      in_specs=[
          pl.BlockSpec((_NC, _R, cin), lambda i: (0, i, 0)),
          pl.BlockSpec((_R, cin), lambda i: (i, 0)),
          pl.BlockSpec((2 * cout, cin), lambda i: (0, 0)),
          pl.BlockSpec((1, cout), lambda i: (0, 0)),
      ],
      out_specs=[
          pl.BlockSpec((_R, cout // 2), lambda i: (i, 0)),
          pl.BlockSpec((_R, cout), lambda i: (i, 0)),
          pl.BlockSpec((1, 8, 128), lambda i: (i, 0, 0)),
      ],
      out_shape=[
          jax.ShapeDtypeStruct((_N, cout // 2), jnp.int32),
          jax.ShapeDtypeStruct((_N, cout), jnp.float32),
          jax.ShapeDtypeStruct((_NB, 8, 128), jnp.float32),
      ],
  )(aggp, r_prev, wcat, b_rel.reshape(1, -1))


def _pool_and_heads(aggp, r_prev, batch3, w1s, b1s, w2s, b2s, w3s, b3s,
                    wos, bos):
  def body(a_ref, rp_ref, bt_ref, w1_ref, b1_ref, w2_ref, b2_ref,
           w3_ref, b3_ref, wo_ref, bo_ref, out_ref, pool_ref, cnt_ref):
    i = pl.program_id(0)

    @pl.when(i == 0)
    def _init():
      pool_ref[...] = jnp.zeros_like(pool_ref)
      cnt_ref[...] = jnp.zeros_like(cnt_ref)

    h = jnp.maximum(a_ref[0] + a_ref[1] + rp_ref[...], 0.0)
    labels = lax.broadcasted_iota(jnp.int32, (_G, _R), 0)
    onehot = (labels == bt_ref[0]).astype(jnp.float32)
    pool_ref[...] += jnp.dot(onehot, h, preferred_element_type=jnp.float32)
    cnt_ref[:, 0:1] += jnp.sum(onehot, axis=1, keepdims=True)

    @pl.when(i == _NB - 1)
    def _heads():
      pooled = pool_ref[...] / jnp.maximum(cnt_ref[:, 0:1], 1.0)
      cols = []
      for hd in range(_NCLS):
        hc = jnp.maximum(
            jnp.dot(pooled, w1_ref[hd].T,
                    preferred_element_type=jnp.float32) + b1_ref[hd], 0.0)
        hc = jnp.maximum(
            jnp.dot(hc, w2_ref[hd].T,
                    preferred_element_type=jnp.float32) + b2_ref[hd], 0.0)
        hc = jnp.maximum(
            jnp.dot(hc, w3_ref[hd].T,
                    preferred_element_type=jnp.float32) + b3_ref[hd], 0.0)
        o = jnp.dot(hc, wo_ref[hd].reshape(-1, 1),
                    preferred_element_type=jnp.float32) + bo_ref[0, hd]
        cols.append(o)
      out_ref[...] = jnp.concatenate(cols, axis=1)

  full = lambda s: pl.BlockSpec(s, lambda i: tuple(0 for _ in s))
  return pl.pallas_call(
      body,
      grid=(_NB,),
      in_specs=[
          pl.BlockSpec((_NC, _R, 64), lambda i: (0, i, 0)),
          pl.BlockSpec((_R, 64), lambda i: (i, 0)),
          pl.BlockSpec((1, 1, _R), lambda i: (i, 0, 0)),
          full(w1s.shape), full(b1s.shape), full(w2s.shape), full(b2s.shape),
          full(w3s.shape), full(b3s.shape), full(wos.shape), full(bos.shape),
      ],
      out_specs=pl.BlockSpec((_G, _NCLS), lambda i: (0, 0)),
      out_shape=jax.ShapeDtypeStruct((_G, _NCLS), jnp.float32),
      scratch_shapes=[
          pltpu.VMEM((_G, 64), jnp.float32),
          pltpu.VMEM((_G, 128), jnp.float32),
      ],
  )(aggp, r_prev, batch3, w1s, b1s, w2s, b2s, w3s, b3s, wos, bos)


@jax.jit
def kernel(x, edge_index, batch, edge_attr, params):
  src = edge_index[0]
  dst = edge_index[1]
  pad = _EPAD - _E
  src_p = jnp.concatenate([src, jnp.zeros((pad,), jnp.int32)])
  src_p = src_p.reshape(_EPAD // _CHUNK, _CHUNK)
  dst_p = jnp.concatenate([dst, jnp.zeros((pad,), jnp.int32)])
  dst_p = dst_p.reshape(_EPAD // _CHUNK, _CHUNK)
  w_p = jnp.concatenate([edge_attr, jnp.zeros((pad,), jnp.float32)])
  batch3 = batch.reshape(_NB, 1, _R)

  gcn = params['gcn']
  g, r, s_out = _proj_first(x, gcn[0]['W_rel'], gcn[0]['b_rel'],
                            gcn[0]['W_root'])
  for li in range(1, len(gcn)):
    cout_prev = 2 * g.shape[1]
    s16 = jnp.pad(s_out[:, 0, 0], (0, 16 - _NB))
    aggp = _edge_aggregate(cout_prev)(g, src_p, dst_p, w_p, s16)
    g, r, s_out = _proj_mid(aggp, r, gcn[li]['W_rel'], gcn[li]['b_rel'],
                            gcn[li]['W_root'])
  s16 = jnp.pad(s_out[:, 0, 0], (0, 16 - _NB))
  aggp = _edge_aggregate(64)(g, src_p, dst_p, w_p, s16)

  w1s = jnp.stack([m[0]['W'] for m in params['mlp']])
  b1s = jnp.stack([m[0]['b'] for m in params['mlp']])
  w2s = jnp.stack([m[1]['W'] for m in params['mlp']])
  b2s = jnp.stack([m[1]['b'] for m in params['mlp']])
  w3s = jnp.stack([m[2]['W'] for m in params['mlp']])
  b3s = jnp.stack([m[2]['b'] for m in params['mlp']])
  wos = jnp.stack([o['W'].reshape(-1) for o in params['out']])
  bos = jnp.stack([o['b'].reshape(()) for o in params['out']]).reshape(1, -1)

  return _pool_and_heads(aggp, r, batch3, w1s, b1s, w2s, b2s, w3s, b3s,
                         wos, bos)
